# baseline (speedup 1.0000x reference)
import functools

import jax
import jax.numpy as jnp
from jax import lax
from jax.experimental import pallas as pl
from jax.experimental.pallas import tpu as pltpu
from jax.experimental.pallas import tpu_sc as plsc

EMB = 64
HID = 256
L = 50
LP = 64
LANES = 16
NC = 2
NS = 16
NW = NC * NS
NBUF = 4
KV = LP // LANES
EMBP = 128


def _sc_pool(table, idx2, n_rows):
    rw = n_rows // NW
    mesh = plsc.VectorSubcoreMesh(
        core_axis_name="c", subcore_axis_name="s",
        num_cores=NC, num_subcores=NS)

    @functools.partial(
        pl.kernel,
        out_type=jax.ShapeDtypeStruct((n_rows, EMB), jnp.float32),
        mesh=mesh,
        scratch_types=(
            [pltpu.VMEM((rw, LP), jnp.int32)]
            + [pltpu.VMEM((LP, EMBP), jnp.float32) for _ in range(NBUF)]
            + [pltpu.VMEM((rw, EMB), jnp.float32)]
            + [pltpu.SemaphoreType.DMA for _ in range(NBUF)]
        ),
        compiler_params=pltpu.CompilerParams(use_tc_tiling_on_sc=False),
    )
    def body(table_hbm, idx_hbm, out_hbm, idx_v, *rest):
        bufs = rest[:NBUF]
        out_v = rest[NBUF]
        sems = rest[NBUF + 1:]
        wid = lax.axis_index("s") * NC + lax.axis_index("c")
        rbase = wid * rw
        pltpu.sync_copy(idx_hbm.at[pl.ds(rbase, rw)], idx_v)

        def gstart(r, b):
            for k in range(KV):
                iv = idx_v[r, pl.ds(k * LANES, LANES)]
                pltpu.make_async_copy(
                    table_hbm.at[iv],
                    bufs[b].at[pl.ds(k * LANES, LANES)], sems[b]).start()

        def gwait(b):
            iv = idx_v[0, pl.ds(0, LANES)]
            for k in range(KV):
                pltpu.make_async_copy(
                    table_hbm.at[iv],
                    bufs[b].at[pl.ds(k * LANES, LANES)], sems[b]).wait()

        def process(r, b):
            buf = bufs[b]
            def tstep(t, acc):
                return tuple(
                    acc[c] + buf[t, pl.ds(c * LANES, LANES)]
                    for c in range(EMB // LANES))
            zero = jnp.zeros((LANES,), jnp.float32)
            acc = lax.fori_loop(0, LP, tstep,
                                (zero,) * (EMB // LANES), unroll=4)
            for c in range(EMB // LANES):
                out_v[r, pl.ds(c * LANES, LANES)] = acc[c]

        for b in range(NBUF):
            gstart(b, b)

        def step(i, carry):
            r0 = NBUF * i
            for b in range(NBUF):
                gwait(b)
                process(r0 + b, b)
                gstart(r0 + b + NBUF, b)
            return carry

        lax.fori_loop(0, rw // NBUF - 1, step, 0)
        for b in range(NBUF):
            gwait(b)
            process(rw - NBUF + b, b)

        pltpu.sync_copy(out_v, out_hbm.at[pl.ds(rbase, rw)])

    return body(table, idx2)


def _tc_towers(sums3, idx3, t0, w1s, b1s, w2s, b2s):
    nb = sums3.shape[1]

    def body(x_ref, idx_ref, t0_ref, w1_ref, b1_ref, w2_ref, b2_ref, o_ref):
        idx = idx_ref[0]
        cnt = jnp.sum((idx != 0).astype(jnp.float32), axis=1, keepdims=True)
        c0 = jnp.float32(LP) - cnt
        x = x_ref[0] - c0 * t0_ref[...]
        inv = jnp.where(cnt > 0, 1.0 / jnp.maximum(cnt, 1.0), 0.0)
        x = x * inv
        h = jnp.maximum(
            jnp.dot(x, w1_ref[0], preferred_element_type=jnp.float32,
                    precision=lax.Precision.HIGHEST) + b1_ref[0], 0.0)
        o_ref[0] = jnp.maximum(
            jnp.dot(h, w2_ref[0], preferred_element_type=jnp.float32,
                    precision=lax.Precision.HIGHEST) + b2_ref[0], 0.0)

    return pl.pallas_call(
        body,
        grid=(3,),
        in_specs=[
            pl.BlockSpec((1, nb, EMB), lambda i: (i, 0, 0)),
            pl.BlockSpec((1, nb, L), lambda i: (i, 0, 0)),
            pl.BlockSpec((1, EMB), lambda i: (0, 0)),
            pl.BlockSpec((1, EMB, HID), lambda i: (i, 0, 0)),
            pl.BlockSpec((1, 1, HID), lambda i: (i, 0, 0)),
            pl.BlockSpec((1, HID, EMB), lambda i: (i, 0, 0)),
            pl.BlockSpec((1, 1, EMB), lambda i: (i, 0, 0)),
        ],
        out_specs=pl.BlockSpec((1, nb, EMB), lambda i: (i, 0, 0)),
        out_shape=jax.ShapeDtypeStruct((3, nb, EMB), jnp.float32),
    )(sums3, idx3, t0, w1s, b1s, w2s, b2s)


def kernel(query_input, pos_answer_input, neg_answer_input, table,
           qW1, qb1, qW2, qb2, aW1, ab1, aW2, ab2):
    nb = query_input.shape[0]
    n_rows = 3 * nb
    idx = jnp.concatenate(
        [query_input, pos_answer_input, neg_answer_input], axis=0)
    idx2 = jnp.pad(idx, ((0, 0), (0, LP - L)))

    tablep = jnp.pad(table, ((0, 0), (0, EMBP - EMB)))
    sums = _sc_pool(tablep, idx2, n_rows)
    sums3 = sums.reshape(3, nb, EMB)
    idx3 = idx.reshape(3, nb, L)

    t0 = table[0:1]
    w1s = jnp.stack([qW1, aW1, aW1])
    b1s = jnp.stack([qb1, ab1, ab1]).reshape(3, 1, HID)
    w2s = jnp.stack([qW2, aW2, aW2])
    b2s = jnp.stack([qb2, ab2, ab2]).reshape(3, 1, EMB)

    out = _tc_towers(sums3, idx3, t0, w1s, b1s, w2s, b2s)
    return (out[0], out[1], out[2])


# speedup vs baseline: 7.6198x; 7.6198x over previous
import functools

import numpy as np
import jax
import jax.numpy as jnp
from jax import lax
from jax.experimental import pallas as pl
from jax.experimental.pallas import tpu as pltpu
from jax.experimental.pallas import tpu_sc as plsc

EMB = 64
HID = 256
L = 50
LANES = 16
NC = 2
NS = 16
NW = NC * NS
CH = 4
NBUF = 2
GRP = CH * L

PERM = np.concatenate(
    [g * 32 + np.concatenate([np.arange(0, 32, 2), np.arange(1, 32, 2)])
     for g in range(EMB // 32)])


def _sc_pool(tableb, idx2, n_rows):
    ng = n_rows // CH // NW
    rw = n_rows // NW
    mesh = plsc.VectorSubcoreMesh(
        core_axis_name="c", subcore_axis_name="s",
        num_cores=NC, num_subcores=NS)

    @functools.partial(
        pl.kernel,
        out_type=jax.ShapeDtypeStruct((n_rows, EMB), jnp.float32),
        mesh=mesh,
        scratch_types=(
            [pltpu.VMEM((ng, GRP), jnp.int32)]
            + [pltpu.VMEM((GRP, EMB), jnp.bfloat16) for _ in range(NBUF)]
            + [pltpu.VMEM((rw, EMB), jnp.float32)]
            + [pltpu.SemaphoreType.DMA for _ in range(NBUF)]
        ),
        compiler_params=pltpu.CompilerParams(
            use_tc_tiling_on_sc=False, needs_layout_passes=False),
    )
    def body(table_hbm, idx_hbm, out_hbm, idx_v, *rest):
        bufs = rest[:NBUF]
        out_v = rest[NBUF]
        sems = rest[NBUF + 1:]
        wid = lax.axis_index("s") * NC + lax.axis_index("c")
        rbase = wid * rw
        pltpu.sync_copy(idx_hbm.at[pl.ds(wid * ng, ng)], idx_v)

        def gstart(g, b):
            pltpu.make_async_copy(
                table_hbm.at[idx_v.at[g]], bufs[b], sems[b]).start()

        def gwait(b):
            pltpu.make_async_copy(
                table_hbm.at[idx_v.at[0]], bufs[b], sems[b]).wait()

        def process(g, b):
            buf = bufs[b]
            hi_mask = jnp.full((LANES,), -65536, jnp.int32)
            for r in range(CH):
                def tstep(t, acc):
                    row = r * L + t
                    w0 = plsc.bitcast(buf[row, pl.ds(0, 32)], jnp.int32)
                    w1 = plsc.bitcast(buf[row, pl.ds(32, 32)], jnp.int32)
                    return (
                        acc[0] + plsc.bitcast(w0 << 16, jnp.float32),
                        acc[1] + plsc.bitcast(w0 & hi_mask, jnp.float32),
                        acc[2] + plsc.bitcast(w1 << 16, jnp.float32),
                        acc[3] + plsc.bitcast(w1 & hi_mask, jnp.float32),
                    )
                zero = jnp.zeros((LANES,), jnp.float32)
                acc = lax.fori_loop(0, L, tstep, (zero,) * 4, unroll=4)
                for c in range(4):
                    out_v[g * CH + r, pl.ds(c * LANES, LANES)] = acc[c]

        for b in range(NBUF):
            gstart(b, b)

        def step(i, carry):
            g0 = NBUF * i
            for b in range(NBUF):
                gwait(b)
                process(g0 + b, b)
                gstart(g0 + b + NBUF, b)
            return carry

        lax.fori_loop(0, ng // NBUF - 1, step, 0)
        for b in range(NBUF):
            gwait(b)
            process(ng - NBUF + b, b)

        pltpu.sync_copy(out_v, out_hbm.at[pl.ds(rbase, rw)])

    return body(tableb, idx2)


def _tc_towers(sums3, idx3, t0p, w1s, b1s, w2s, b2s):
    nb = sums3.shape[1]

    def body(x_ref, idx_ref, t0_ref, w1_ref, b1_ref, w2_ref, b2_ref, o_ref):
        idx = idx_ref[0]
        cnt = jnp.sum((idx != 0).astype(jnp.float32), axis=1, keepdims=True)
        c0 = jnp.float32(L) - cnt
        x = x_ref[0] - c0 * t0_ref[...]
        inv = jnp.where(cnt > 0, 1.0 / jnp.maximum(cnt, 1.0), 0.0)
        x = x * inv
        h = jnp.maximum(
            jnp.dot(x, w1_ref[0], preferred_element_type=jnp.float32,
                    precision=lax.Precision.HIGHEST) + b1_ref[0], 0.0)
        o_ref[0] = jnp.maximum(
            jnp.dot(h, w2_ref[0], preferred_element_type=jnp.float32,
                    precision=lax.Precision.HIGHEST) + b2_ref[0], 0.0)

    return pl.pallas_call(
        body,
        grid=(3,),
        in_specs=[
            pl.BlockSpec((1, nb, EMB), lambda i: (i, 0, 0)),
            pl.BlockSpec((1, nb, L), lambda i: (i, 0, 0)),
            pl.BlockSpec((1, EMB), lambda i: (0, 0)),
            pl.BlockSpec((1, EMB, HID), lambda i: (i, 0, 0)),
            pl.BlockSpec((1, 1, HID), lambda i: (i, 0, 0)),
            pl.BlockSpec((1, HID, EMB), lambda i: (i, 0, 0)),
            pl.BlockSpec((1, 1, EMB), lambda i: (i, 0, 0)),
        ],
        out_specs=pl.BlockSpec((1, nb, EMB), lambda i: (i, 0, 0)),
        out_shape=jax.ShapeDtypeStruct((3, nb, EMB), jnp.float32),
    )(sums3, idx3, t0p, w1s, b1s, w2s, b2s)


def kernel(query_input, pos_answer_input, neg_answer_input, table,
           qW1, qb1, qW2, qb2, aW1, ab1, aW2, ab2):
    nb = query_input.shape[0]
    n_rows = 3 * nb
    idx = jnp.concatenate(
        [query_input, pos_answer_input, neg_answer_input], axis=0)
    idx2 = idx.reshape(n_rows // CH, GRP)

    tableb = table.astype(jnp.bfloat16)
    sums = _sc_pool(tableb, idx2, n_rows)
    sums3 = sums.reshape(3, nb, EMB)
    idx3 = idx.reshape(3, nb, L)

    perm = jnp.asarray(PERM)
    t0p = tableb[0:1].astype(jnp.float32)[:, perm]
    w1s = jnp.stack([qW1, aW1, aW1])[:, perm, :]
    b1s = jnp.stack([qb1, ab1, ab1]).reshape(3, 1, HID)
    w2s = jnp.stack([qW2, aW2, aW2])
    b2s = jnp.stack([qb2, ab2, ab2]).reshape(3, 1, EMB)

    out = _tc_towers(sums3, idx3, t0p, w1s, b1s, w2s, b2s)
    return (out[0], out[1], out[2])


# speedup vs baseline: 9.5853x; 1.2579x over previous
import functools

import numpy as np
import jax
import jax.numpy as jnp
from jax import lax
from jax.experimental import pallas as pl
from jax.experimental.pallas import tpu as pltpu
from jax.experimental.pallas import tpu_sc as plsc

EMB = 64
HID = 256
L = 50
LANES = 16
NC = 2
NS = 16
NW = NC * NS
CH = 4
NBUF = 2
GRP = CH * L

PERM = np.concatenate(
    [g * 32 + np.concatenate([np.arange(0, 32, 2), np.arange(1, 32, 2)])
     for g in range(EMB // 32)])


def _sc_pool(tableb, idx2, n_rows):
    ng = n_rows // CH // NW
    rw = n_rows // NW
    mesh = plsc.VectorSubcoreMesh(
        core_axis_name="c", subcore_axis_name="s",
        num_cores=NC, num_subcores=NS)

    @functools.partial(
        pl.kernel,
        out_type=jax.ShapeDtypeStruct((n_rows, EMB), jnp.float32),
        mesh=mesh,
        scratch_types=(
            [pltpu.VMEM((ng, GRP), jnp.int32)]
            + [pltpu.VMEM((GRP, EMB), jnp.float32) for _ in range(NBUF)]
            + [pltpu.VMEM((rw, EMB), jnp.float32)]
            + [pltpu.SemaphoreType.DMA for _ in range(NBUF)]
        ),
        compiler_params=pltpu.CompilerParams(
            use_tc_tiling_on_sc=False, needs_layout_passes=False),
    )
    def body(table_hbm, idx_hbm, out_hbm, idx_v, *rest):
        bufs = rest[:NBUF]
        out_v = rest[NBUF]
        sems = rest[NBUF + 1:]
        wid = lax.axis_index("s") * NC + lax.axis_index("c")
        rbase = wid * rw
        pltpu.sync_copy(idx_hbm.at[pl.ds(wid * ng, ng)], idx_v)

        def gstart(g, b):
            pltpu.make_async_copy(
                table_hbm.at[idx_v.at[g]], bufs[b], sems[b]).start()

        def gwait(b):
            pltpu.make_async_copy(
                table_hbm.at[idx_v.at[0]], bufs[b], sems[b]).wait()

        def process(g, b):
            buf = bufs[b]
            for r in range(CH):
                def tstep(t, acc):
                    row = r * L + t
                    return tuple(
                        acc[c] + buf[row, pl.ds(c * LANES, LANES)]
                        for c in range(4))
                zero = jnp.zeros((LANES,), jnp.float32)
                acc = lax.fori_loop(0, L, tstep, (zero,) * 4, unroll=4)
                for c in range(4):
                    out_v[g * CH + r, pl.ds(c * LANES, LANES)] = acc[c]

        for b in range(NBUF):
            gstart(b, b)

        def step(i, carry):
            g0 = NBUF * i
            for b in range(NBUF):
                gwait(b)
                process(g0 + b, b)
                gstart(g0 + b + NBUF, b)
            return carry

        lax.fori_loop(0, ng // NBUF - 1, step, 0)
        for b in range(NBUF):
            gwait(b)
            process(ng - NBUF + b, b)

        pltpu.sync_copy(out_v, out_hbm.at[pl.ds(rbase, rw)])

    return body(tableb, idx2)


def _tc_towers(sums3, idx3, t0p, w1s, b1s, w2s, b2s):
    nb = sums3.shape[1]

    def body(x_ref, idx_ref, t0_ref, w1_ref, b1_ref, w2_ref, b2_ref, o_ref):
        idx = idx_ref[0]
        cnt = jnp.sum((idx != 0).astype(jnp.float32), axis=1, keepdims=True)
        c0 = jnp.float32(L) - cnt
        x = x_ref[0] - c0 * t0_ref[...]
        inv = jnp.where(cnt > 0, 1.0 / jnp.maximum(cnt, 1.0), 0.0)
        x = x * inv
        h = jnp.maximum(
            jnp.dot(x, w1_ref[0], preferred_element_type=jnp.float32,
                    precision=lax.Precision.HIGHEST) + b1_ref[0], 0.0)
        o_ref[0] = jnp.maximum(
            jnp.dot(h, w2_ref[0], preferred_element_type=jnp.float32,
                    precision=lax.Precision.HIGHEST) + b2_ref[0], 0.0)

    return pl.pallas_call(
        body,
        grid=(3,),
        in_specs=[
            pl.BlockSpec((1, nb, EMB), lambda i: (i, 0, 0)),
            pl.BlockSpec((1, nb, L), lambda i: (i, 0, 0)),
            pl.BlockSpec((1, EMB), lambda i: (0, 0)),
            pl.BlockSpec((1, EMB, HID), lambda i: (i, 0, 0)),
            pl.BlockSpec((1, 1, HID), lambda i: (i, 0, 0)),
            pl.BlockSpec((1, HID, EMB), lambda i: (i, 0, 0)),
            pl.BlockSpec((1, 1, EMB), lambda i: (i, 0, 0)),
        ],
        out_specs=pl.BlockSpec((1, nb, EMB), lambda i: (i, 0, 0)),
        out_shape=jax.ShapeDtypeStruct((3, nb, EMB), jnp.float32),
    )(sums3, idx3, t0p, w1s, b1s, w2s, b2s)


def kernel(query_input, pos_answer_input, neg_answer_input, table,
           qW1, qb1, qW2, qb2, aW1, ab1, aW2, ab2):
    nb = query_input.shape[0]
    n_rows = 3 * nb
    idx = jnp.concatenate(
        [query_input, pos_answer_input, neg_answer_input], axis=0)
    idx2 = idx.reshape(n_rows // CH, GRP)

    sums = _sc_pool(table, idx2, n_rows)
    sums3 = sums.reshape(3, nb, EMB)
    idx3 = idx.reshape(3, nb, L)

    t0p = table[0:1]
    w1s = jnp.stack([qW1, aW1, aW1])
    b1s = jnp.stack([qb1, ab1, ab1]).reshape(3, 1, HID)
    w2s = jnp.stack([qW2, aW2, aW2])
    b2s = jnp.stack([qb2, ab2, ab2]).reshape(3, 1, EMB)

    out = _tc_towers(sums3, idx3, t0p, w1s, b1s, w2s, b2s)
    return (out[0], out[1], out[2])
